# baseline (speedup 1.0000x reference)
import functools

import jax
import jax.numpy as jnp
from jax import lax
from jax.experimental import pallas as pl
from jax.experimental.pallas import tpu as pltpu
from jax.experimental.pallas import tpu_sc as plsc

B = 1024
CTX = 50
DIM = 16
VOCAB = 100000

NC = 2
NS = 16
NW = NC * NS
ROWS_W = B // NW
IDX_W = ROWS_W * CTX
CHUNK = 100
NCHUNK = IDX_W // CHUNK


def _sc_gather_sum(inp_flat, emb_table):
    mesh = plsc.VectorSubcoreMesh(
        core_axis_name="c", subcore_axis_name="s", num_cores=NC, num_subcores=NS
    )

    @functools.partial(
        pl.kernel,
        out_type=jax.ShapeDtypeStruct((B, DIM), jnp.float32),
        mesh=mesh,
        scratch_types=[
            pltpu.VMEM((NCHUNK, CHUNK), jnp.int32),
            pltpu.VMEM((IDX_W, DIM), jnp.float32),
            pltpu.VMEM((ROWS_W, DIM), jnp.float32),
            pltpu.SemaphoreType.DMA,
        ],
        compiler_params=pltpu.CompilerParams(use_tc_tiling_on_sc=False),
    )
    def sc_kernel(inp_hbm, table_hbm, out_hbm, idx_v, rows_v, x_v, sem):
        wid = lax.axis_index("s") * NC + lax.axis_index("c")
        pltpu.sync_copy(inp_hbm.at[wid], idx_v)
        copies = [
            pltpu.async_copy(
                table_hbm.at[idx_v.at[j]],
                rows_v.at[pl.ds(j * CHUNK, CHUNK)],
                sem,
            )
            for j in range(NCHUNK)
        ]
        for c in copies:
            c.wait()
        for r in range(ROWS_W):
            def ctx_body(c, acc):
                return acc + rows_v[r * CTX + c, :]
            acc = lax.fori_loop(0, CTX, ctx_body, jnp.zeros((DIM,), jnp.float32),
                                unroll=10)
            x_v[r, :] = acc
        pltpu.sync_copy(x_v, out_hbm.at[pl.ds(wid * ROWS_W, ROWS_W)])

    return sc_kernel(inp_flat, emb_table)


VBLK = 2048
NVBLK = pl.cdiv(VOCAB, VBLK)


def _tc_matmul_t(xt, wt, b2):
    LAST = VOCAB - (NVBLK - 1) * VBLK

    def mm(xt_ref, w_ref, o_hbm, buf, sem):
        i = pl.program_id(0)
        acc = lax.dot_general(
            w_ref[...], xt_ref[...],
            (((0,), (0,)), ((), ())),
            preferred_element_type=jnp.float32,
        )

        def for_slot(s):
            @pl.when(lax.rem(i, 2) == s)
            def _():
                @pl.when(i >= 2)
                def _():
                    pltpu.make_async_copy(
                        buf.at[s], o_hbm.at[pl.ds((i - 2) * VBLK, VBLK)],
                        sem.at[s]).wait()
                buf[s] = acc
                @pl.when(i < NVBLK - 1)
                def _():
                    pltpu.async_copy(
                        buf.at[s], o_hbm.at[pl.ds(i * VBLK, VBLK)], sem.at[s])
                @pl.when(i == NVBLK - 1)
                def _():
                    pltpu.async_copy(
                        buf.at[s, pl.ds(0, LAST)],
                        o_hbm.at[pl.ds(i * VBLK, LAST)], sem.at[s])
                    pltpu.make_async_copy(
                        buf.at[1 - s], o_hbm.at[pl.ds((i - 1) * VBLK, VBLK)],
                        sem.at[1 - s]).wait()
                    pltpu.make_async_copy(
                        buf.at[s, pl.ds(0, LAST)],
                        o_hbm.at[pl.ds(i * VBLK, LAST)], sem.at[s]).wait()

        for_slot(0)
        for_slot(1)

    xt_aug = jnp.concatenate(
        [xt, jnp.ones((1, B), xt.dtype)], axis=0).astype(jnp.bfloat16)
    wt_aug = jnp.concatenate([wt, b2], axis=0).astype(jnp.bfloat16)
    return pl.pallas_call(
        mm,
        grid=(NVBLK,),
        in_specs=[
            pl.BlockSpec((DIM + 1, B), lambda i: (0, 0)),
            pl.BlockSpec((DIM + 1, VBLK), lambda i: (0, i)),
        ],
        out_specs=pl.BlockSpec(memory_space=pl.ANY),
        out_shape=jax.ShapeDtypeStruct((VOCAB, B), jnp.float32),
        scratch_shapes=[
            pltpu.VMEM((2, VBLK, B), jnp.float32),
            pltpu.SemaphoreType.DMA((2,)),
        ],
        compiler_params=pltpu.CompilerParams(
            vmem_limit_bytes=128 * 1024 * 1024,
            dimension_semantics=("arbitrary",),
        ),
    )(xt_aug, wt_aug)


def kernel(inp, emb_table, W, b):
    inp_flat = inp.reshape(NW, NCHUNK, CHUNK).astype(jnp.int32)
    x = _sc_gather_sum(inp_flat, emb_table)
    yt = _tc_matmul_t(x.T, W.T, b.reshape(1, VOCAB))
    return yt.T


# speedup vs baseline: 1.0204x; 1.0204x over previous
import functools

import jax
import jax.numpy as jnp
from jax import lax
from jax.experimental import pallas as pl
from jax.experimental.pallas import tpu as pltpu
from jax.experimental.pallas import tpu_sc as plsc

B = 1024
CTX = 50
DIM = 16
VOCAB = 100000

NC = 2
NS = 16
NW = NC * NS
ROWS_W = B // NW
IDX_W = ROWS_W * CTX
CHUNK = 100
NCHUNK = IDX_W // CHUNK


def _sc_gather_sum(inp_flat, emb_table):
    mesh = plsc.VectorSubcoreMesh(
        core_axis_name="c", subcore_axis_name="s", num_cores=NC, num_subcores=NS
    )

    @functools.partial(
        pl.kernel,
        out_type=jax.ShapeDtypeStruct((B, DIM), jnp.float32),
        mesh=mesh,
        scratch_types=[
            pltpu.VMEM((NCHUNK, CHUNK), jnp.int32),
            pltpu.VMEM((IDX_W, DIM), jnp.float32),
            pltpu.VMEM((ROWS_W, DIM), jnp.float32),
            pltpu.SemaphoreType.DMA,
        ],
        compiler_params=pltpu.CompilerParams(use_tc_tiling_on_sc=False),
    )
    def sc_kernel(inp_hbm, table_hbm, out_hbm, idx_v, rows_v, x_v, sem):
        wid = lax.axis_index("s") * NC + lax.axis_index("c")
        pltpu.sync_copy(inp_hbm.at[wid], idx_v)
        copies = [
            pltpu.async_copy(
                table_hbm.at[idx_v.at[j]],
                rows_v.at[pl.ds(j * CHUNK, CHUNK)],
                sem,
            )
            for j in range(NCHUNK)
        ]
        for c in copies:
            c.wait()
        for r in range(ROWS_W):
            def ctx_body(c, acc):
                return acc + rows_v[r * CTX + c, :]
            acc = lax.fori_loop(0, CTX, ctx_body, jnp.zeros((DIM,), jnp.float32),
                                unroll=10)
            x_v[r, :] = acc
        pltpu.sync_copy(x_v, out_hbm.at[pl.ds(wid * ROWS_W, ROWS_W)])

    return sc_kernel(inp_flat, emb_table)


VBLK = 2048
NVBLK = pl.cdiv(VOCAB, VBLK)


def _tc_matmul_t(xt, wt, b2):
    def mm(xt_ref, w_ref, b_ref, o_ref):
        acc = lax.dot_general(
            w_ref[...].astype(jnp.bfloat16), xt_ref[...].astype(jnp.bfloat16),
            (((0,), (0,)), ((), ())),
            preferred_element_type=jnp.float32,
        )
        o_ref[...] = acc + jnp.transpose(b_ref[...])

    return pl.pallas_call(
        mm,
        grid=(NVBLK,),
        in_specs=[
            pl.BlockSpec((DIM, B), lambda i: (0, 0)),
            pl.BlockSpec((DIM, VBLK), lambda i: (0, i)),
            pl.BlockSpec((1, VBLK), lambda i: (0, i)),
        ],
        out_specs=pl.BlockSpec((VBLK, B), lambda i: (i, 0)),
        out_shape=jax.ShapeDtypeStruct((VOCAB, B), jnp.float32),
    )(xt, wt, b2)


def kernel(inp, emb_table, W, b):
    inp_flat = inp.reshape(NW, NCHUNK, CHUNK).astype(jnp.int32)
    x = _sc_gather_sum(inp_flat, emb_table)
    yt = _tc_matmul_t(x.T, W.T, b.reshape(1, VOCAB))
    return yt.T
